# Initial kernel scaffold; baseline (speedup 1.0000x reference)
import jax
import jax.numpy as jnp
from jax.experimental import pallas as pl


def kernel(input, t_p):
    raise NotImplementedError("write your pallas kernel here")


# speedup vs baseline: 4.8155x; 4.8155x over previous
import jax
import jax.numpy as jnp
from jax.experimental import pallas as pl
from jax.experimental.pallas import tpu as pltpu

_H = 14
_W = 14
_S = _H * _W


def _body(x_ref, tp_ref, masked_ref, tmpl_ref):
    x = x_ref[0]
    s, c = x.shape
    mx = jnp.max(x, axis=0, keepdims=True)
    iota = jax.lax.broadcasted_iota(jnp.int32, (s, c), 0)
    idx = jnp.min(jnp.where(x >= mx, iota, s), axis=0, keepdims=True)
    onehot = (iota == idx).astype(jnp.float32)
    tmpl = jax.lax.dot_general(
        tp_ref[...], onehot,
        dimension_numbers=(((0,), (0,)), ((), ())),
        preferred_element_type=jnp.float32,
    )
    tmpl_ref[0] = tmpl
    masked_ref[0] = jnp.maximum(x * tmpl, 0.0)


def kernel(input, t_p):
    x = input
    b, h, w, c = x.shape
    s = h * w
    x3 = x.reshape(b, s, c)
    tp2 = t_p.reshape(s, s)
    masked, tmpl = pl.pallas_call(
        _body,
        grid=(b,),
        in_specs=[
            pl.BlockSpec((1, s, c), lambda i: (i, 0, 0)),
            pl.BlockSpec((s, s), lambda i: (0, 0)),
        ],
        out_specs=[
            pl.BlockSpec((1, s, c), lambda i: (i, 0, 0)),
            pl.BlockSpec((1, s, c), lambda i: (i, 0, 0)),
        ],
        out_shape=[
            jax.ShapeDtypeStruct((b, s, c), jnp.float32),
            jax.ShapeDtypeStruct((b, s, c), jnp.float32),
        ],
    )(x3, tp2)
    return (masked.reshape(b, h, w, c), x, tmpl.reshape(b, h, w, c))
